# Initial kernel scaffold; baseline (speedup 1.0000x reference)
import jax
import jax.numpy as jnp
from jax.experimental import pallas as pl


def kernel(token_ids, w):
    raise NotImplementedError("write your pallas kernel here")


# speedup vs baseline: 1.7477x; 1.7477x over previous
import functools

import jax
import jax.numpy as jnp
from jax import lax
from jax.experimental import pallas as pl
from jax.experimental.pallas import tpu as pltpu
from jax.experimental.pallas import tpu_sc as plsc

_BATCH = 16384
_HIST = 50
_DIM = 64

_B = _BATCH * _HIST
_NC = 2
_NS = 16
_NW = _NC * _NS
_BPW = _B // _NW
_CHUNK = 128
_NCHUNK = _BPW // _CHUNK


@functools.partial(
    pl.kernel,
    out_type=jax.ShapeDtypeStruct((_B, _DIM), jnp.float32),
    mesh=plsc.VectorSubcoreMesh(core_axis_name="c", subcore_axis_name="s"),
    compiler_params=pltpu.CompilerParams(use_tc_tiling_on_sc=False),
    scratch_types=[
        pltpu.VMEM((_NCHUNK, _CHUNK), jnp.int32),
        pltpu.VMEM((_CHUNK, _DIM), jnp.float32),
        pltpu.VMEM((_CHUNK, _DIM), jnp.float32),
        pltpu.SemaphoreType.DMA,
        pltpu.SemaphoreType.DMA,
    ],
)
def _sc_gather(table_hbm, idx_hbm, out_hbm, idx_v, rows0, rows1, gsem0, gsem1):
    wid = lax.axis_index("s") * _NC + lax.axis_index("c")
    base = wid * _BPW

    pltpu.sync_copy(idx_hbm.at[pl.ds(wid * _NCHUNK, _NCHUNK)], idx_v)

    rows = (rows0, rows1)
    gsem = (gsem0, gsem1)
    last = _NCHUNK - 1

    def gather_start(g, b):
        pltpu.async_copy(table_hbm.at[idx_v.at[g]], rows[b], gsem[b])

    def gather_wait(g, b):
        pltpu.make_async_copy(table_hbm.at[idx_v.at[g]], rows[b], gsem[b]).wait()

    gather_start(0, 0)

    def body(i, carry):
        for b in range(2):
            g = 2 * i + b
            gather_wait(g, b)
            gather_start(jnp.minimum(g + 1, last), 1 - b)
            pltpu.sync_copy(rows[b], out_hbm.at[pl.ds(base + g * _CHUNK, _CHUNK)])
        return carry

    lax.fori_loop(0, _NCHUNK // 2, body, 0)
    gather_wait(last, 0)


def kernel(token_ids, w):
    idx = token_ids.astype(jnp.int32).reshape(_B // _CHUNK, _CHUNK)
    out = _sc_gather(w, idx)
    return out.reshape(_BATCH, _HIST, _DIM)
